# Initial kernel scaffold; baseline (speedup 1.0000x reference)
import jax
import jax.numpy as jnp
from jax.experimental import pallas as pl


def kernel(x, emb, W1, b1, W2, b2, W3, b3):
    raise NotImplementedError("write your pallas kernel here")


# speedup vs baseline: 7.6232x; 7.6232x over previous
import functools

import jax
import jax.numpy as jnp
from jax import lax
from jax.experimental import pallas as pl
from jax.experimental.pallas import tpu as pltpu
from jax.experimental.pallas import tpu_sc as plsc

VOCAB = 100000
EMB = 128
HID = 1024
OUT = 32
B = 4096
L = 50

NC = 2
NS = 16
NW = NC * NS
BW = B // NW
CB = 8
NCHUNK = BW // CB
LANES = 16
KV = EMB // LANES


def _pool_body(xflat_hbm, emb_hbm, out_hbm, idx_v, rows_v, pool_v, sem):
    wid = lax.axis_index("s") * NC + lax.axis_index("c")
    base = wid * BW

    def chunk(c, carry):
        rbase = base + c * CB
        pltpu.sync_copy(xflat_hbm.at[pl.ds(rbase * L, CB * L)], idx_v)
        pltpu.async_copy(emb_hbm.at[idx_v], rows_v, sem).wait()
        for b in range(CB):
            def red(j, accs):
                return tuple(
                    accs[k] + rows_v[b * L + j, pl.ds(k * LANES, LANES)]
                    for k in range(KV)
                )
            accs = lax.fori_loop(
                0, L, red, tuple(jnp.zeros((LANES,), jnp.float32) for _ in range(KV))
            )
            for k in range(KV):
                pool_v[b, pl.ds(k * LANES, LANES)] = accs[k] * (1.0 / L)
        pltpu.sync_copy(pool_v, out_hbm.at[pl.ds(rbase, CB)])
        return carry

    lax.fori_loop(0, NCHUNK, chunk, 0)


@functools.partial(
    pl.kernel,
    out_type=jax.ShapeDtypeStruct((B, EMB), jnp.float32),
    mesh=plsc.VectorSubcoreMesh(core_axis_name="c", subcore_axis_name="s"),
    scratch_types=[
        pltpu.VMEM((CB * L,), jnp.int32),
        pltpu.VMEM((CB * L, EMB), jnp.float32),
        pltpu.VMEM((CB, EMB), jnp.float32),
        pltpu.SemaphoreType.DMA,
    ],
)
def _pool(xflat_hbm, emb_hbm, out_hbm, idx_v, rows_v, pool_v, sem):
    _pool_body(xflat_hbm, emb_hbm, out_hbm, idx_v, rows_v, pool_v, sem)


BM = 512


def _mlp_body(h_ref, w1_ref, b1_ref, w2_ref, b2_ref, w3_ref, b3_ref, o_ref):
    h = h_ref[...]
    h1 = jnp.dot(h, w1_ref[...], preferred_element_type=jnp.float32)
    h1 = jnp.maximum(h1 + b1_ref[...], 0.0)
    h2 = jnp.dot(h1, w2_ref[...], preferred_element_type=jnp.float32)
    h2 = jnp.maximum(h2 + b2_ref[...], 0.0)
    o_ref[...] = jnp.dot(h2, w3_ref[...], preferred_element_type=jnp.float32) + b3_ref[...]


def _mlp(pooled, w1t, b1, w2t, b2, w3t, b3):
    grid = (B // BM,)
    return pl.pallas_call(
        _mlp_body,
        grid=grid,
        in_specs=[
            pl.BlockSpec((BM, EMB), lambda i: (i, 0)),
            pl.BlockSpec((EMB, HID), lambda i: (0, 0)),
            pl.BlockSpec((1, HID), lambda i: (0, 0)),
            pl.BlockSpec((HID, HID // 2), lambda i: (0, 0)),
            pl.BlockSpec((1, HID // 2), lambda i: (0, 0)),
            pl.BlockSpec((HID // 2, OUT), lambda i: (0, 0)),
            pl.BlockSpec((1, OUT), lambda i: (0, 0)),
        ],
        out_specs=pl.BlockSpec((BM, OUT), lambda i: (i, 0)),
        out_shape=jax.ShapeDtypeStruct((B, OUT), jnp.float32),
    )(pooled, w1t, b1, w2t, b2, w3t, b3)


def kernel(x, emb, W1, b1, W2, b2, W3, b3):
    xflat = x.reshape(-1).astype(jnp.int32)
    pooled = _pool(xflat, emb)
    return _mlp(
        pooled,
        W1.T, b1.reshape(1, HID),
        W2.T, b2.reshape(1, HID // 2),
        W3.T, b3.reshape(1, OUT),
    )


# speedup vs baseline: 10.7318x; 1.4078x over previous
import functools

import jax
import jax.numpy as jnp
from jax import lax
from jax.experimental import pallas as pl
from jax.experimental.pallas import tpu as pltpu
from jax.experimental.pallas import tpu_sc as plsc

VOCAB = 100000
EMB = 128
HID = 1024
OUT = 32
B = 4096
L = 50

NC = 2
NS = 16
NW = NC * NS
BW = B // NW
CB = 8
NCHUNK = BW // CB
LANES = 16
KV = EMB // LANES


def _pool_body(xflat_hbm, emb_hbm, out_hbm, idx_a, idx_b, rows_a, rows_b,
               pool_v, sem_a, sem_b):
    wid = lax.axis_index("s") * NC + lax.axis_index("c")
    base = wid * BW

    def start(c, idx_v, rows_v, sem):
        pltpu.sync_copy(xflat_hbm.at[pl.ds((base + c * CB) * L, CB * L)], idx_v)
        pltpu.async_copy(emb_hbm.at[idx_v], rows_v, sem)

    def drain(idx_v, rows_v, sem):
        pltpu.make_async_copy(emb_hbm.at[idx_v], rows_v, sem).wait()

    def reduce_chunk(c, rows_v):
        rbase = base + c * CB
        for b in range(CB):
            def red(j, accs):
                r = rows_v
                return tuple(
                    accs[k] + r[b * L + 2 * j, pl.ds(k * LANES, LANES)]
                    + r[b * L + 2 * j + 1, pl.ds(k * LANES, LANES)]
                    for k in range(KV)
                )
            accs = lax.fori_loop(
                0, L // 2, red,
                tuple(jnp.zeros((LANES,), jnp.float32) for _ in range(KV)),
            )
            for k in range(KV):
                pool_v[b, pl.ds(k * LANES, LANES)] = accs[k] * (1.0 / L)
        pltpu.sync_copy(pool_v, out_hbm.at[pl.ds(rbase, CB)])

    start(0, idx_a, rows_a, sem_a)

    def pair(g, carry):
        c_a = 2 * g
        start(c_a + 1, idx_b, rows_b, sem_b)
        drain(idx_a, rows_a, sem_a)
        reduce_chunk(c_a, rows_a)

        @pl.when(g < NCHUNK // 2 - 1)
        def _():
            start(c_a + 2, idx_a, rows_a, sem_a)

        drain(idx_b, rows_b, sem_b)
        reduce_chunk(c_a + 1, rows_b)
        return carry

    lax.fori_loop(0, NCHUNK // 2, pair, 0)


@functools.partial(
    pl.kernel,
    out_type=jax.ShapeDtypeStruct((B, EMB), jnp.float32),
    mesh=plsc.VectorSubcoreMesh(core_axis_name="c", subcore_axis_name="s"),
    scratch_types=[
        pltpu.VMEM((CB * L,), jnp.int32),
        pltpu.VMEM((CB * L,), jnp.int32),
        pltpu.VMEM((CB * L, EMB), jnp.float32),
        pltpu.VMEM((CB * L, EMB), jnp.float32),
        pltpu.VMEM((CB, EMB), jnp.float32),
        pltpu.SemaphoreType.DMA,
        pltpu.SemaphoreType.DMA,
    ],
)
def _pool(xflat_hbm, emb_hbm, out_hbm, idx_a, idx_b, rows_a, rows_b,
          pool_v, sem_a, sem_b):
    _pool_body(xflat_hbm, emb_hbm, out_hbm, idx_a, idx_b, rows_a, rows_b,
               pool_v, sem_a, sem_b)


BM = 512


def _mlp_body(h_ref, w1_ref, b1_ref, w2_ref, b2_ref, w3_ref, b3_ref, o_ref):
    h = h_ref[...]
    h1 = jnp.dot(h, w1_ref[...], preferred_element_type=jnp.float32)
    h1 = jnp.maximum(h1 + b1_ref[...], 0.0)
    h2 = jnp.dot(h1, w2_ref[...], preferred_element_type=jnp.float32)
    h2 = jnp.maximum(h2 + b2_ref[...], 0.0)
    o_ref[...] = jnp.dot(h2, w3_ref[...], preferred_element_type=jnp.float32) + b3_ref[...]


def _mlp(pooled, w1t, b1, w2t, b2, w3t, b3):
    grid = (B // BM,)
    return pl.pallas_call(
        _mlp_body,
        grid=grid,
        in_specs=[
            pl.BlockSpec((BM, EMB), lambda i: (i, 0)),
            pl.BlockSpec((EMB, HID), lambda i: (0, 0)),
            pl.BlockSpec((1, HID), lambda i: (0, 0)),
            pl.BlockSpec((HID, HID // 2), lambda i: (0, 0)),
            pl.BlockSpec((1, HID // 2), lambda i: (0, 0)),
            pl.BlockSpec((HID // 2, OUT), lambda i: (0, 0)),
            pl.BlockSpec((1, OUT), lambda i: (0, 0)),
        ],
        out_specs=pl.BlockSpec((BM, OUT), lambda i: (i, 0)),
        out_shape=jax.ShapeDtypeStruct((B, OUT), jnp.float32),
    )(pooled, w1t, b1, w2t, b2, w3t, b3)


def kernel(x, emb, W1, b1, W2, b2, W3, b3):
    xflat = x.reshape(-1).astype(jnp.int32)
    pooled = _pool(xflat, emb)
    return _mlp(
        pooled,
        W1.T, b1.reshape(1, HID),
        W2.T, b2.reshape(1, HID // 2),
        W3.T, b3.reshape(1, OUT),
    )
